# baseline (speedup 1.0000x reference)
import functools

import jax
import jax.numpy as jnp
from jax import lax
from jax.experimental import pallas as pl
from jax.experimental.pallas import tpu as pltpu
from jax.experimental.pallas import tpu_sc as plsc

_NC = 2
_NS = 16
_NW = _NC * _NS
_CHUNK = 128
_PAD = 128
_NPAD = 10240
_NCHUNK = 2


def _sc_mesh():
    return plsc.VectorSubcoreMesh(core_axis_name="c", subcore_axis_name="s")


def _sc_gather(table, idx):
    n, d = table.shape
    e = idx.shape[0]
    n_chunks = e // _CHUNK
    per_w = -(-n_chunks // _NW)

    @functools.partial(
        pl.kernel,
        mesh=_sc_mesh(),
        out_type=jax.ShapeDtypeStruct((e, d), table.dtype),
        scratch_types=[
            pltpu.VMEM((_CHUNK,), jnp.int32),
            pltpu.VMEM((_CHUNK, d), table.dtype),
            pltpu.SemaphoreType.DMA,
        ],
    )
    def k(table_hbm, idx_hbm, out_hbm, idx_v, rows_v, sem):
        wid = lax.axis_index("s") * _NC + lax.axis_index("c")

        def body(j, carry):
            c = wid * per_w + j

            @pl.when(c < n_chunks)
            def _():
                base = c * _CHUNK
                pltpu.sync_copy(idx_hbm.at[pl.ds(base, _CHUNK)], idx_v)
                pltpu.async_copy(table_hbm.at[idx_v], rows_v, sem).wait()
                pltpu.sync_copy(rows_v, out_hbm.at[pl.ds(base, _CHUNK)])

            return carry

        lax.fori_loop(0, per_w, body, 0)

    return k(table, idx)


def _sc_scatter_add(rows, idx, zeros_blk):
    e, d = rows.shape
    n_chunks = e // _CHUNK
    per_w = -(-n_chunks // _NW)
    rpt = _NPAD // _NS

    @functools.partial(
        pl.kernel,
        mesh=_sc_mesh(),
        out_type=jax.ShapeDtypeStruct((_NC, _NPAD, d), jnp.float32),
        scratch_types=[
            pltpu.VMEM((_CHUNK,), jnp.int32),
            pltpu.VMEM((_CHUNK, d), jnp.float32),
            pltpu.VMEM((_CHUNK, d), jnp.float32),
            pltpu.VMEM_SHARED((_NPAD, d), jnp.float32),
        ],
    )
    def k(rows_hbm, idx_hbm, zeros_hbm, out_hbm, idx_v, rows_v, zeros_v,
          acc_sh):
        cid = lax.axis_index("c")
        sid = lax.axis_index("s")
        wid = sid * _NC + cid
        pltpu.sync_copy(zeros_hbm, zeros_v)

        def zbody(j, carry):
            pltpu.sync_copy(
                zeros_v, acc_sh.at[pl.ds(sid * rpt + j * _CHUNK, _CHUNK)]
            )
            return carry

        lax.fori_loop(0, rpt // _CHUNK, zbody, 0)
        plsc.subcore_barrier()

        def body(j, carry):
            c = wid * per_w + j

            @pl.when(c < n_chunks)
            def _():
                base = c * _CHUNK
                pltpu.sync_copy(idx_hbm.at[pl.ds(base, _CHUNK)], idx_v)
                pltpu.sync_copy(rows_hbm.at[pl.ds(base, _CHUNK)], rows_v)
                pltpu.sync_copy(rows_v, acc_sh.at[idx_v], add=True)

            return carry

        lax.fori_loop(0, per_w, body, 0)
        plsc.subcore_barrier()
        pltpu.sync_copy(
            acc_sh.at[pl.ds(sid * rpt, rpt)],
            out_hbm.at[cid, pl.ds(sid * rpt, rpt)],
        )

    return k(rows, idx, zeros_blk)


def _msg_kernel(ea_ref, g_ref, w1_ref, b1_ref, m2_ref, s_ref, r_ref, bb_ref,
                out_ref):
    hh = bb_ref.shape[1]
    he = jnp.maximum(
        jnp.dot(ea_ref[...], w1_ref[...], preferred_element_type=jnp.float32)
        + b1_ref[...],
        0.0,
    )
    g = g_ref[...][:, :hh]
    tmp = jnp.dot(g, m2_ref[...], preferred_element_type=jnp.float32)
    a = jnp.dot(he, s_ref[...], preferred_element_type=jnp.float32)
    msg = jnp.dot(a * tmp, r_ref[...], preferred_element_type=jnp.float32)
    msg = msg + jnp.dot(g, bb_ref[...], preferred_element_type=jnp.float32)
    eb = msg.shape[0]
    pad = jnp.concatenate(
        [
            jnp.ones((eb, 1), jnp.float32),
            jnp.zeros((eb, _PAD - hh - 1), jnp.float32),
        ],
        axis=1,
    )
    out_ref[...] = jnp.concatenate([msg, pad], axis=1)


def _tc_messages(edge_attr, g, w1, b1r, m2, s, r, bb):
    e = edge_attr.shape[0]
    eb = 1600
    return pl.pallas_call(
        _msg_kernel,
        grid=(e // eb,),
        in_specs=[
            pl.BlockSpec((eb, edge_attr.shape[1]), lambda i: (i, 0)),
            pl.BlockSpec((eb, _PAD), lambda i: (i, 0)),
            pl.BlockSpec(w1.shape, lambda i: (0, 0)),
            pl.BlockSpec(b1r.shape, lambda i: (0, 0)),
            pl.BlockSpec(m2.shape, lambda i: (0, 0)),
            pl.BlockSpec(s.shape, lambda i: (0, 0)),
            pl.BlockSpec(r.shape, lambda i: (0, 0)),
            pl.BlockSpec(bb.shape, lambda i: (0, 0)),
        ],
        out_specs=pl.BlockSpec((eb, _PAD), lambda i: (i, 0)),
        out_shape=jax.ShapeDtypeStruct((e, _PAD), jnp.float32),
    )(edge_attr, g, w1, b1r, m2, s, r, bb)


def _update_kernel(h_ref, *rest):
    p_refs = rest[:-3]
    root_ref, b_ref, out_ref = rest[-3:]
    hh = root_ref.shape[1]
    acc = p_refs[0][0] + p_refs[0][1]
    for p in p_refs[1:]:
        acc = acc + p[0] + p[1]
    aggr = acc[:, :hh]
    deg = jnp.maximum(acc[:, hh : hh + 1], 1.0)
    val = jnp.maximum(
        jnp.dot(h_ref[...], root_ref[...], preferred_element_type=jnp.float32)
        + aggr / deg
        + b_ref[...],
        0.0,
    )
    nb = val.shape[0]
    out_ref[...] = jnp.concatenate(
        [val, jnp.zeros((nb, _PAD - hh), jnp.float32)], axis=1
    )


def _tc_update(h, ps, root_pad, biasr):
    n = h.shape[0]
    nb = 2000
    return pl.pallas_call(
        _update_kernel,
        grid=(n // nb,),
        in_specs=[pl.BlockSpec((nb, _PAD), lambda i: (i, 0))]
        + [pl.BlockSpec((2, nb, _PAD), lambda i: (0, i, 0)) for _ in ps]
        + [
            pl.BlockSpec(root_pad.shape, lambda i: (0, 0)),
            pl.BlockSpec(biasr.shape, lambda i: (0, 0)),
        ],
        out_specs=pl.BlockSpec((nb, _PAD), lambda i: (i, 0)),
        out_shape=jax.ShapeDtypeStruct((n, _PAD), jnp.float32),
    )(h, *ps, root_pad, biasr)


def _set2set_kernel(nb, h_ref, bid_ref, wi_ref, wh_ref, bl_ref, lw_ref, lb_ref,
                    out_ref):
    hh = wh_ref.shape[0]
    h = h_ref[...][:, :hh]
    bid = bid_ref[...]
    cols = lax.broadcasted_iota(jnp.int32, (1, nb), 1)
    mask = bid == cols
    maskf = mask.astype(jnp.float32)
    q_star = jnp.zeros((nb, 2 * hh), jnp.float32)
    hx = jnp.zeros((nb, hh), jnp.float32)
    cx = jnp.zeros((nb, hh), jnp.float32)
    for _ in range(3):
        gates = (
            jnp.dot(q_star, wi_ref[...], preferred_element_type=jnp.float32)
            + jnp.dot(hx, wh_ref[...], preferred_element_type=jnp.float32)
            + bl_ref[...]
        )
        i_g = jax.nn.sigmoid(gates[:, 0 * hh : 1 * hh])
        f_g = jax.nn.sigmoid(gates[:, 1 * hh : 2 * hh])
        g_g = jnp.tanh(gates[:, 2 * hh : 3 * hh])
        o_g = jax.nn.sigmoid(gates[:, 3 * hh : 4 * hh])
        cx = f_g * cx + i_g * g_g
        hx = o_g * jnp.tanh(cx)
        q = hx
        qb = jnp.dot(maskf, q, preferred_element_type=jnp.float32)
        e = jnp.sum(h * qb, axis=1, keepdims=True)
        em = jnp.where(mask, e, -1e30)
        e_max = jnp.max(em, axis=0, keepdims=True)
        a = jnp.where(mask, jnp.exp(e - e_max), 0.0)
        denom = jnp.maximum(jnp.sum(a, axis=0, keepdims=True), 1e-30)
        attn = a / denom
        r = lax.dot_general(
            attn, h, (((0,), (0,)), ((), ())), preferred_element_type=jnp.float32
        )
        q_star = jnp.concatenate([q, r], axis=1)
    out_ref[...] = (
        jnp.dot(q_star, lw_ref[...], preferred_element_type=jnp.float32)
        + lb_ref[...]
    )


def _tc_set2set(h, bid, wi, wh, blr, lw, lbr, nb):
    t = lw.shape[1]
    return pl.pallas_call(
        functools.partial(_set2set_kernel, nb),
        out_shape=jax.ShapeDtypeStruct((nb, t), jnp.float32),
    )(h, bid, wi, wh, blr, lw, lbr)


def kernel(x, edge_index, edge_attr, batch, W1, b1, W2, b2, root, conv_bias,
           Wi, Wh, b_lstm, lin_w, lin_b):
    n, f_node = x.shape
    e = edge_attr.shape[0]
    h_dim = W1.shape[1]
    b_graphs = 16
    src = edge_index[0]
    dst = edge_index[1]

    m2 = (
        W2.reshape(h_dim, f_node, h_dim)
        .transpose(1, 0, 2)
        .reshape(f_node, h_dim * h_dim)
    )
    s = jnp.kron(jnp.eye(h_dim, dtype=jnp.float32), jnp.ones((1, h_dim), jnp.float32))
    r = jnp.tile(jnp.eye(h_dim, dtype=jnp.float32), (h_dim, 1))
    bb = b2.reshape(f_node, h_dim)
    b1r = b1.reshape(1, h_dim)
    biasr = conv_bias.reshape(1, h_dim)
    root_pad = jnp.concatenate(
        [root, jnp.zeros((_PAD - f_node, h_dim), jnp.float32)], axis=0
    )
    zeros_blk = jnp.zeros((_CHUNK, _PAD), jnp.float32)

    ec = e // _NCHUNK
    srcs = [lax.slice(src, (i * ec,), ((i + 1) * ec,)) for i in range(_NCHUNK)]
    dsts = [lax.slice(dst, (i * ec,), ((i + 1) * ec,)) for i in range(_NCHUNK)]
    eas = [
        lax.slice(edge_attr, (i * ec, 0), ((i + 1) * ec, edge_attr.shape[1]))
        for i in range(_NCHUNK)
    ]

    h = jnp.concatenate([x, jnp.zeros((n, _PAD - f_node), jnp.float32)], axis=1)
    for _ in range(3):
        gs = [_sc_gather(h, srcs[i]) for i in range(_NCHUNK)]
        msgs = [
            _tc_messages(eas[i], gs[i], W1, b1r, m2, s, r, bb)
            for i in range(_NCHUNK)
        ]
        ps = [
            _sc_scatter_add(msgs[i], dsts[i], zeros_blk)
            for i in range(_NCHUNK)
        ]
        h = _tc_update(h, ps, root_pad, biasr)

    return _tc_set2set(
        h,
        batch.reshape(n, 1),
        Wi,
        Wh,
        b_lstm.reshape(1, 4 * h_dim),
        lin_w,
        lin_b.reshape(1, lin_w.shape[1]),
        b_graphs,
    )


# speedup vs baseline: 1.0752x; 1.0752x over previous
import functools

import jax
import jax.numpy as jnp
from jax import lax
from jax.experimental import pallas as pl
from jax.experimental.pallas import tpu as pltpu
from jax.experimental.pallas import tpu_sc as plsc

_NC = 2
_NS = 16
_NW = _NC * _NS
_CHUNK = 128
_PAD = 128
_NPAD = 10240
_NCHUNK = 5


def _sc_mesh():
    return plsc.VectorSubcoreMesh(core_axis_name="c", subcore_axis_name="s")


def _sc_gather(table, idx):
    n, d = table.shape
    e = idx.shape[0]
    n_chunks = e // _CHUNK
    per_w = -(-n_chunks // _NW)

    @functools.partial(
        pl.kernel,
        mesh=_sc_mesh(),
        out_type=jax.ShapeDtypeStruct((e, d), table.dtype),
        scratch_types=[
            pltpu.VMEM((_CHUNK,), jnp.int32),
            pltpu.VMEM((_CHUNK, d), table.dtype),
            pltpu.SemaphoreType.DMA,
        ],
    )
    def k(table_hbm, idx_hbm, out_hbm, idx_v, rows_v, sem):
        wid = lax.axis_index("s") * _NC + lax.axis_index("c")

        def body(j, carry):
            c = wid * per_w + j

            @pl.when(c < n_chunks)
            def _():
                base = c * _CHUNK
                pltpu.sync_copy(idx_hbm.at[pl.ds(base, _CHUNK)], idx_v)
                pltpu.async_copy(table_hbm.at[idx_v], rows_v, sem).wait()
                pltpu.sync_copy(rows_v, out_hbm.at[pl.ds(base, _CHUNK)])

            return carry

        lax.fori_loop(0, per_w, body, 0)

    return k(table, idx)


def _sc_scatter_add(rows, idx, zeros_blk):
    e, d = rows.shape
    n_chunks = e // _CHUNK
    per_w = -(-n_chunks // _NW)
    rpt = _NPAD // _NS

    @functools.partial(
        pl.kernel,
        mesh=_sc_mesh(),
        out_type=jax.ShapeDtypeStruct((_NC, _NPAD, d), jnp.float32),
        scratch_types=[
            pltpu.VMEM((_CHUNK,), jnp.int32),
            pltpu.VMEM((_CHUNK, d), jnp.float32),
            pltpu.VMEM((_CHUNK, d), jnp.float32),
            pltpu.VMEM_SHARED((_NPAD, d), jnp.float32),
        ],
    )
    def k(rows_hbm, idx_hbm, zeros_hbm, out_hbm, idx_v, rows_v, zeros_v,
          acc_sh):
        cid = lax.axis_index("c")
        sid = lax.axis_index("s")
        wid = sid * _NC + cid
        pltpu.sync_copy(zeros_hbm, zeros_v)

        def zbody(j, carry):
            pltpu.sync_copy(
                zeros_v, acc_sh.at[pl.ds(sid * rpt + j * _CHUNK, _CHUNK)]
            )
            return carry

        lax.fori_loop(0, rpt // _CHUNK, zbody, 0)
        plsc.subcore_barrier()

        def body(j, carry):
            c = wid * per_w + j

            @pl.when(c < n_chunks)
            def _():
                base = c * _CHUNK
                pltpu.sync_copy(idx_hbm.at[pl.ds(base, _CHUNK)], idx_v)
                pltpu.sync_copy(rows_hbm.at[pl.ds(base, _CHUNK)], rows_v)
                pltpu.sync_copy(rows_v, acc_sh.at[idx_v], add=True)

            return carry

        lax.fori_loop(0, per_w, body, 0)
        plsc.subcore_barrier()
        pltpu.sync_copy(
            acc_sh.at[pl.ds(sid * rpt, rpt)],
            out_hbm.at[cid, pl.ds(sid * rpt, rpt)],
        )

    return k(rows, idx, zeros_blk)


def _msg_kernel(ea_ref, g_ref, w1_ref, b1_ref, m2_ref, s_ref, r_ref, bb_ref,
                out_ref):
    hh = bb_ref.shape[1]
    he = jnp.maximum(
        jnp.dot(ea_ref[...], w1_ref[...], preferred_element_type=jnp.float32)
        + b1_ref[...],
        0.0,
    )
    g = g_ref[...][:, :hh]
    tmp = jnp.dot(g, m2_ref[...], preferred_element_type=jnp.float32)
    a = jnp.dot(he, s_ref[...], preferred_element_type=jnp.float32)
    msg = jnp.dot(a * tmp, r_ref[...], preferred_element_type=jnp.float32)
    msg = msg + jnp.dot(g, bb_ref[...], preferred_element_type=jnp.float32)
    eb = msg.shape[0]
    pad = jnp.concatenate(
        [
            jnp.ones((eb, 1), jnp.float32),
            jnp.zeros((eb, _PAD - hh - 1), jnp.float32),
        ],
        axis=1,
    )
    out_ref[...] = jnp.concatenate([msg, pad], axis=1)


def _tc_messages(edge_attr, g, w1, b1r, m2, s, r, bb):
    e = edge_attr.shape[0]
    eb = 1600
    return pl.pallas_call(
        _msg_kernel,
        grid=(e // eb,),
        in_specs=[
            pl.BlockSpec((eb, edge_attr.shape[1]), lambda i: (i, 0)),
            pl.BlockSpec((eb, _PAD), lambda i: (i, 0)),
            pl.BlockSpec(w1.shape, lambda i: (0, 0)),
            pl.BlockSpec(b1r.shape, lambda i: (0, 0)),
            pl.BlockSpec(m2.shape, lambda i: (0, 0)),
            pl.BlockSpec(s.shape, lambda i: (0, 0)),
            pl.BlockSpec(r.shape, lambda i: (0, 0)),
            pl.BlockSpec(bb.shape, lambda i: (0, 0)),
        ],
        out_specs=pl.BlockSpec((eb, _PAD), lambda i: (i, 0)),
        out_shape=jax.ShapeDtypeStruct((e, _PAD), jnp.float32),
    )(edge_attr, g, w1, b1r, m2, s, r, bb)


def _update_kernel(h_ref, *rest):
    p_refs = rest[:-3]
    root_ref, b_ref, out_ref = rest[-3:]
    hh = root_ref.shape[1]
    acc = p_refs[0][0] + p_refs[0][1]
    for p in p_refs[1:]:
        acc = acc + p[0] + p[1]
    aggr = acc[:, :hh]
    deg = jnp.maximum(acc[:, hh : hh + 1], 1.0)
    val = jnp.maximum(
        jnp.dot(h_ref[...], root_ref[...], preferred_element_type=jnp.float32)
        + aggr / deg
        + b_ref[...],
        0.0,
    )
    nb = val.shape[0]
    out_ref[...] = jnp.concatenate(
        [val, jnp.zeros((nb, _PAD - hh), jnp.float32)], axis=1
    )


def _tc_update(h, ps, root_pad, biasr):
    n = h.shape[0]
    nb = 2000
    return pl.pallas_call(
        _update_kernel,
        grid=(n // nb,),
        in_specs=[pl.BlockSpec((nb, _PAD), lambda i: (i, 0))]
        + [pl.BlockSpec((2, nb, _PAD), lambda i: (0, i, 0)) for _ in ps]
        + [
            pl.BlockSpec(root_pad.shape, lambda i: (0, 0)),
            pl.BlockSpec(biasr.shape, lambda i: (0, 0)),
        ],
        out_specs=pl.BlockSpec((nb, _PAD), lambda i: (i, 0)),
        out_shape=jax.ShapeDtypeStruct((n, _PAD), jnp.float32),
    )(h, *ps, root_pad, biasr)


def _set2set_kernel(nb, h_ref, bid_ref, wi_ref, wh_ref, bl_ref, lw_ref, lb_ref,
                    out_ref):
    hh = wh_ref.shape[0]
    h = h_ref[...][:, :hh]
    bid = bid_ref[...]
    cols = lax.broadcasted_iota(jnp.int32, (1, nb), 1)
    mask = bid == cols
    maskf = mask.astype(jnp.float32)
    q_star = jnp.zeros((nb, 2 * hh), jnp.float32)
    hx = jnp.zeros((nb, hh), jnp.float32)
    cx = jnp.zeros((nb, hh), jnp.float32)
    for _ in range(3):
        gates = (
            jnp.dot(q_star, wi_ref[...], preferred_element_type=jnp.float32)
            + jnp.dot(hx, wh_ref[...], preferred_element_type=jnp.float32)
            + bl_ref[...]
        )
        i_g = jax.nn.sigmoid(gates[:, 0 * hh : 1 * hh])
        f_g = jax.nn.sigmoid(gates[:, 1 * hh : 2 * hh])
        g_g = jnp.tanh(gates[:, 2 * hh : 3 * hh])
        o_g = jax.nn.sigmoid(gates[:, 3 * hh : 4 * hh])
        cx = f_g * cx + i_g * g_g
        hx = o_g * jnp.tanh(cx)
        q = hx
        qb = jnp.dot(maskf, q, preferred_element_type=jnp.float32)
        e = jnp.sum(h * qb, axis=1, keepdims=True)
        em = jnp.where(mask, e, -1e30)
        e_max = jnp.max(em, axis=0, keepdims=True)
        a = jnp.where(mask, jnp.exp(e - e_max), 0.0)
        denom = jnp.maximum(jnp.sum(a, axis=0, keepdims=True), 1e-30)
        attn = a / denom
        r = lax.dot_general(
            attn, h, (((0,), (0,)), ((), ())), preferred_element_type=jnp.float32
        )
        q_star = jnp.concatenate([q, r], axis=1)
    out_ref[...] = (
        jnp.dot(q_star, lw_ref[...], preferred_element_type=jnp.float32)
        + lb_ref[...]
    )


def _tc_set2set(h, bid, wi, wh, blr, lw, lbr, nb):
    t = lw.shape[1]
    return pl.pallas_call(
        functools.partial(_set2set_kernel, nb),
        out_shape=jax.ShapeDtypeStruct((nb, t), jnp.float32),
    )(h, bid, wi, wh, blr, lw, lbr)


def kernel(x, edge_index, edge_attr, batch, W1, b1, W2, b2, root, conv_bias,
           Wi, Wh, b_lstm, lin_w, lin_b):
    n, f_node = x.shape
    e = edge_attr.shape[0]
    h_dim = W1.shape[1]
    b_graphs = 16
    src = edge_index[0]
    dst = edge_index[1]

    m2 = (
        W2.reshape(h_dim, f_node, h_dim)
        .transpose(1, 0, 2)
        .reshape(f_node, h_dim * h_dim)
    )
    s = jnp.kron(jnp.eye(h_dim, dtype=jnp.float32), jnp.ones((1, h_dim), jnp.float32))
    r = jnp.tile(jnp.eye(h_dim, dtype=jnp.float32), (h_dim, 1))
    bb = b2.reshape(f_node, h_dim)
    b1r = b1.reshape(1, h_dim)
    biasr = conv_bias.reshape(1, h_dim)
    root_pad = jnp.concatenate(
        [root, jnp.zeros((_PAD - f_node, h_dim), jnp.float32)], axis=0
    )
    zeros_blk = jnp.zeros((_CHUNK, _PAD), jnp.float32)

    ec = e // _NCHUNK
    srcs = [lax.slice(src, (i * ec,), ((i + 1) * ec,)) for i in range(_NCHUNK)]
    dsts = [lax.slice(dst, (i * ec,), ((i + 1) * ec,)) for i in range(_NCHUNK)]
    eas = [
        lax.slice(edge_attr, (i * ec, 0), ((i + 1) * ec, edge_attr.shape[1]))
        for i in range(_NCHUNK)
    ]

    h = jnp.concatenate([x, jnp.zeros((n, _PAD - f_node), jnp.float32)], axis=1)
    for _ in range(3):
        gs = [_sc_gather(h, srcs[i]) for i in range(_NCHUNK)]
        msgs = [
            _tc_messages(eas[i], gs[i], W1, b1r, m2, s, r, bb)
            for i in range(_NCHUNK)
        ]
        ps = [
            _sc_scatter_add(msgs[i], dsts[i], zeros_blk)
            for i in range(_NCHUNK)
        ]
        h = _tc_update(h, ps, root_pad, biasr)

    return _tc_set2set(
        h,
        batch.reshape(n, 1),
        Wi,
        Wh,
        b_lstm.reshape(1, 4 * h_dim),
        lin_w,
        lin_b.reshape(1, lin_w.shape[1]),
        b_graphs,
    )
